# baseline (speedup 1.0000x reference)
import functools

import jax
import jax.numpy as jnp
from jax import lax
from jax.experimental import pallas as pl
from jax.experimental.pallas import tpu as pltpu
from jax.experimental.pallas import tpu_sc as plsc

NUM_UNITS = 768
NUM_REL = 129


def _zero_row0_body(table_ref, out_ref):
    rows = lax.broadcasted_iota(jnp.int32, table_ref.shape, 0)
    out_ref[...] = jnp.where(rows == 0, jnp.float32(0.0), table_ref[...])


def _zero_row0(table):
    return pl.pallas_call(
        _zero_row0_body,
        out_shape=jax.ShapeDtypeStruct(table.shape, table.dtype),
    )(table)


@functools.lru_cache(maxsize=None)
def _make_sc_gather(B, D):
    info = plsc.get_sparse_core_info()
    NC, NS = info.num_cores, info.num_subcores
    NW = NC * NS
    b_per_w = B // NW
    CH = 16
    ISTAGE = 2048
    n_stage = b_per_w // ISTAGE
    nch = ISTAGE // CH
    assert b_per_w % ISTAGE == 0 and ISTAGE % CH == 0 and nch % 2 == 0

    mesh = plsc.VectorSubcoreMesh(core_axis_name="c", subcore_axis_name="s")

    @functools.partial(
        pl.kernel,
        mesh=mesh,
        out_type=jax.ShapeDtypeStruct((B * D,), jnp.float32),
        scratch_types=[
            pltpu.VMEM((NUM_REL * D,), jnp.float32),
            pltpu.VMEM((ISTAGE,), jnp.int32),
            pltpu.VMEM((CH * D,), jnp.float32),
            pltpu.VMEM((CH * D,), jnp.float32),
            pltpu.SemaphoreType.DMA,
            pltpu.SemaphoreType.DMA,
        ],
    )
    def gather_kernel(table_hbm, idx_hbm, out_hbm, table_v, idx_v,
                      buf0, buf1, wsem0, wsem1):
        wid = lax.axis_index("s") * NC + lax.axis_index("c")
        base = wid * b_per_w

        pltpu.sync_copy(table_hbm, table_v)

        buf = (buf0, buf1)
        wsem = (wsem0, wsem1)

        def w_copy(row0, b):
            return pltpu.make_async_copy(
                buf[b], out_hbm.at[pl.ds(row0 * D, CH * D)], wsem[b])

        def stage_body(si):
            sbase = base + si * ISTAGE
            pltpu.sync_copy(idx_hbm.at[pl.ds(sbase, ISTAGE)], idx_v)

            def chunk_pair(g):
                for b in range(2):
                    t = g + b

                    @pl.when(t >= 2)
                    def _():
                        w_copy(sbase + (t - 2) * CH, b).wait()

                    iv = idx_v[pl.ds(t * CH, CH)]
                    fbs = [iv[r] * D for r in range(CH)]

                    def ubody(u, b=b, fbs=fbs):
                        for r in range(CH):
                            buf[b][pl.ds(r * D + u * 16, 16)] = (
                                table_v[pl.ds(fbs[r] + u * 16, 16)])

                    pl.loop(0, D // 16)(ubody)
                    w_copy(sbase + t * CH, b).start()

            pl.loop(0, nch, step=2)(chunk_pair)
            w_copy(sbase + (nch - 2) * CH, 0).wait()
            w_copy(sbase + (nch - 1) * CH, 1).wait()

        pl.loop(0, n_stage)(stage_body)

    return gather_kernel


def kernel(relation_matrix, embeddings_table):
    bsz, seq, seq2 = relation_matrix.shape
    num_units = embeddings_table.shape[1]
    idx = relation_matrix.reshape(-1)
    table = _zero_row0(embeddings_table).reshape(-1)
    out = _make_sc_gather(idx.shape[0], num_units)(table, idx)
    return out.reshape(bsz, seq, seq2, num_units)


# speedup vs baseline: 2.1778x; 2.1778x over previous
import functools

import jax
import jax.numpy as jnp
from jax import lax
from jax.experimental import pallas as pl
from jax.experimental.pallas import tpu as pltpu
from jax.experimental.pallas import tpu_sc as plsc

NUM_UNITS = 768
NUM_REL = 129


def _zero_row0_body(table_ref, out_ref):
    rows = lax.broadcasted_iota(jnp.int32, table_ref.shape, 0)
    out_ref[...] = jnp.where(rows == 0, jnp.float32(0.0), table_ref[...])


def _zero_row0(table):
    return pl.pallas_call(
        _zero_row0_body,
        out_shape=jax.ShapeDtypeStruct(table.shape, table.dtype),
    )(table)


@functools.lru_cache(maxsize=None)
def _make_sc_gather(B, D):
    info = plsc.get_sparse_core_info()
    NC, NS = info.num_cores, info.num_subcores
    NW = NC * NS
    b_per_w = B // NW
    C = 32
    NBUF = 4
    AHEAD = 2
    nch = b_per_w // C
    assert b_per_w % C == 0 and nch % NBUF == 0

    mesh = plsc.VectorSubcoreMesh(core_axis_name="c", subcore_axis_name="s")

    @functools.partial(
        pl.kernel,
        mesh=mesh,
        out_type=jax.ShapeDtypeStruct((B, D), jnp.float32),
        scratch_types=(
            [pltpu.VMEM((b_per_w,), jnp.int32)]
            + [pltpu.VMEM((C, D), jnp.float32)] * NBUF
            + [pltpu.SemaphoreType.DMA] * (2 * NBUF)
        ),
    )
    def gather_kernel(table_hbm, idx_hbm, out_hbm, idx_v, *bufs_and_sems):
        rows = bufs_and_sems[:NBUF]
        gsem = bufs_and_sems[NBUF:2 * NBUF]
        wsem = bufs_and_sems[2 * NBUF:]
        wid = lax.axis_index("s") * NC + lax.axis_index("c")
        base = wid * b_per_w
        pltpu.sync_copy(idx_hbm.at[pl.ds(base, b_per_w)], idx_v)

        def g_copy(c, b):
            return pltpu.make_async_copy(
                table_hbm.at[idx_v.at[pl.ds(c * C, C)]], rows[b], gsem[b])

        def w_copy(c, b):
            return pltpu.make_async_copy(
                rows[b], out_hbm.at[pl.ds(base + c * C, C)], wsem[b])

        for k in range(AHEAD):
            g_copy(k, k).start()

        def loop_body(g):
            for b in range(NBUF):
                c = g + b
                g_copy(c, b).wait()
                w_copy(c, b).start()

                @pl.when(c >= AHEAD)
                def _():
                    w_copy(c - AHEAD, (b - AHEAD) % NBUF).wait()

                @pl.when(c + AHEAD < nch)
                def _():
                    g_copy(c + AHEAD, (b + AHEAD) % NBUF).start()

        pl.loop(0, nch, step=NBUF)(loop_body)
        for k in range(AHEAD):
            c = nch - AHEAD + k
            w_copy(c, c % NBUF).wait()

    return gather_kernel


def kernel(relation_matrix, embeddings_table):
    bsz, seq, seq2 = relation_matrix.shape
    num_units = embeddings_table.shape[1]
    idx = relation_matrix.reshape(-1)
    table = _zero_row0(embeddings_table)
    out = _make_sc_gather(idx.shape[0], num_units)(table, idx)
    return out.reshape(bsz, seq, seq2, num_units)
